# baseline (speedup 1.0000x reference)
import functools

import jax
import jax.numpy as jnp
from jax import lax
from jax.experimental import pallas as pl
from jax.experimental.pallas import tpu as pltpu
from jax.experimental.pallas import tpu_sc as plsc

_NC = 2
_NS = 16
_NW = _NC * _NS

_N = 4096
_S = 2048
_ND = 4
_NCONT = 32
_DIM = 512
_NROW = _ND + _NCONT
_HEAD = 8
_TAIL = _NROW - _HEAD
_PW = _N // _NW
_PP = 2
_STEPS = _PW // _PP


def _sc_body(idx_hbm, cont_hbm, dtab_hbm, ctab_hbm, out_hbm,
             idx_v, cont_v, ctab_v, gbuf, abuf, cbuf, gsem, osem0, osem1):
    wid = lax.axis_index("s") * _NC + lax.axis_index("c")
    p0 = wid * _PW
    bsel = p0 // _S
    sbase = p0 % _S
    osem = (osem0, osem1)

    pltpu.sync_copy(idx_hbm.at[pl.ds(p0 * _ND, _PW * _ND)], idx_v)
    pltpu.sync_copy(cont_hbm.at[pl.ds(p0, _PW)], cont_v)
    pltpu.sync_copy(ctab_hbm, ctab_v)

    def do_step(s, nb):
        off = pl.multiple_of(s * (_PP * _ND), 8)
        gh = pltpu.async_copy(dtab_hbm.at[idx_v.at[pl.ds(off, _PP * _ND)]],
                              gbuf.at[nb], gsem)

        def make_ck(jg, scal2):
            def ck(k, c):
                ks = pl.ds(k * 16, 16)
                rows = [ctab_v[jg + t, ks] for t in range(8)]
                for t in range(8):
                    j = jg + t
                    for pp in range(_PP):
                        v = scal2[pp][t] * rows[t]
                        if j < _HEAD - _ND:
                            abuf[nb, pp, _ND + j, ks] = v
                        else:
                            cbuf[nb, pp, j - (_HEAD - _ND), ks] = v
                return c
            return ck

        for jg in range(0, _NCONT, 8):
            scal2 = []
            for pp in range(_PP):
                pos = s * _PP + pp
                half = cont_v[pos, pl.ds((jg // 16) * 16, 16)]
                scal2.append([half[jg % 16 + t] for t in range(8)])
            lax.fori_loop(0, _DIM // 16, make_ck(jg, scal2), 0)

        gh.wait()

        def cpk(k, c):
            ks = pl.ds(k * 16, 16)
            for pp in range(_PP):
                for r in range(_ND):
                    abuf[nb, pp, r, ks] = gbuf[nb, pp * _ND + r, ks]
            return c
        lax.fori_loop(0, _DIM // 16, cpk, 0)

        spos = sbase + s * _PP
        for pp in range(_PP):
            pltpu.async_copy(abuf.at[nb, pp],
                             out_hbm.at[bsel, spos + pp, pl.ds(0, _HEAD)],
                             osem[nb])
            pltpu.async_copy(cbuf.at[nb, pp],
                             out_hbm.at[bsel, spos + pp, pl.ds(_HEAD, _TAIL)],
                             osem[nb])

    def drain(nb):
        pltpu.make_async_copy(out_hbm.at[0, pl.ds(0, _PP), pl.ds(0, _HEAD)],
                              abuf.at[nb], osem[nb]).wait()
        pltpu.make_async_copy(out_hbm.at[0, pl.ds(0, _PP), pl.ds(_HEAD, _TAIL)],
                              cbuf.at[nb], osem[nb]).wait()

    do_step(0, 0)
    do_step(1, 1)

    def outer(s2, c):
        for nb in range(2):
            drain(nb)
            do_step(s2 * 2 + nb, nb)
        return c
    lax.fori_loop(1, _STEPS // 2, outer, 0)
    drain(0)
    drain(1)


@jax.jit
def _sc_call(flat_idx, cont, disc_table, cont_table):
    mesh = plsc.VectorSubcoreMesh(core_axis_name="c", subcore_axis_name="s")
    f = functools.partial(
        pl.kernel, _sc_body, mesh=mesh,
        out_type=jax.ShapeDtypeStruct((_N // _S, _S, _NROW, _DIM), jnp.float32),
        scratch_types=[
            pltpu.VMEM((_PW * _ND,), jnp.int32),
            pltpu.VMEM((_PW, _NCONT), jnp.float32),
            pltpu.VMEM((_NCONT, _DIM), jnp.float32),
            pltpu.VMEM((2, _PP * _ND, _DIM), jnp.float32),
            pltpu.VMEM((2, _PP, _HEAD, _DIM), jnp.float32),
            pltpu.VMEM((2, _PP, _TAIL, _DIM), jnp.float32),
            pltpu.SemaphoreType.DMA,
            pltpu.SemaphoreType.DMA,
            pltpu.SemaphoreType.DMA,
        ],
    )()
    return f(flat_idx, cont, disc_table, cont_table)


def kernel(discrete_actions, continuous_actions, disc_table, cont_table, offsets):
    b, s, n_disc = discrete_actions.shape
    n_cont = continuous_actions.shape[-1]
    dim = disc_table.shape[-1]
    n = b * s
    flat_idx = (discrete_actions + offsets[None, None, :]).reshape(n * n_disc)
    cont = continuous_actions.reshape(n, n_cont)
    out = _sc_call(flat_idx, cont, disc_table, cont_table)
    return out.reshape(b, s, n_disc + n_cont, dim)


# speedup vs baseline: 1.2178x; 1.2178x over previous
import jax
import jax.numpy as jnp
from jax import lax
from jax.experimental import pallas as pl
from jax.experimental.pallas import tpu as pltpu

_NBUF = 3


def _body(idx_ref, cont_ref, disc_tab_ref, cont_tab_ref, out_ref,
          obuf, sem0, sem1, sem2):
    i = pl.program_id(0)
    n_steps = pl.num_programs(0)
    sems = (sem0, sem1, sem2)
    r = idx_ref.shape[0]
    vocab = disc_tab_ref.shape[0]

    idx = idx_ref[...]
    iota = jax.lax.broadcasted_iota(jnp.int32, (r, 4, vocab), 2)
    one_hot = (idx[:, :, None] == iota).astype(jnp.float32)
    disc = jax.lax.dot_general(
        one_hot, disc_tab_ref[...],
        dimension_numbers=(((2,), (0,)), ((), ())),
        preferred_element_type=jnp.float32,
    )
    cont = cont_ref[...][:, :, None] * cont_tab_ref[...][None, :, :]
    slab = jnp.concatenate([disc, cont], axis=1)

    for nb in range(_NBUF):
        @pl.when(lax.rem(i, _NBUF) == nb)
        def _():
            @pl.when(i >= _NBUF)
            def _():
                pltpu.make_async_copy(obuf.at[nb], out_ref.at[pl.ds(0, r)],
                                      sems[nb]).wait()
            obuf[nb] = slab
            pltpu.make_async_copy(obuf.at[nb], out_ref.at[pl.ds(i * r, r)],
                                  sems[nb]).start()

    @pl.when(i == n_steps - 1)
    def _():
        for nb in range(_NBUF):
            pltpu.make_async_copy(obuf.at[nb], out_ref.at[pl.ds(0, r)],
                                  sems[nb]).wait()


def kernel(discrete_actions, continuous_actions, disc_table, cont_table, offsets):
    b, s, n_disc = discrete_actions.shape
    n_cont = continuous_actions.shape[-1]
    dim = disc_table.shape[-1]
    n = b * s
    flat_idx = (discrete_actions + offsets[None, None, :]).reshape(n, n_disc)
    cont = continuous_actions.reshape(n, n_cont)

    R = 128
    nrow = n_disc + n_cont
    grid = (n // R,)
    out = pl.pallas_call(
        _body,
        grid=grid,
        in_specs=[
            pl.BlockSpec((R, n_disc), lambda i: (i, 0)),
            pl.BlockSpec((R, n_cont), lambda i: (i, 0)),
            pl.BlockSpec(disc_table.shape, lambda i: (0, 0)),
            pl.BlockSpec(cont_table.shape, lambda i: (0, 0)),
        ],
        out_specs=pl.BlockSpec(memory_space=pl.ANY),
        out_shape=jax.ShapeDtypeStruct((n, nrow, dim), jnp.float32),
        scratch_shapes=[
            pltpu.VMEM((_NBUF, R, nrow, dim), jnp.float32),
            pltpu.SemaphoreType.DMA,
            pltpu.SemaphoreType.DMA,
            pltpu.SemaphoreType.DMA,
        ],
    )(flat_idx, cont, disc_table, cont_table)
    return out.reshape(b, s, nrow, dim)
